# baseline (speedup 1.0000x reference)
import jax
import jax.numpy as jnp
from jax import lax
from jax.experimental import pallas as pl
from jax.experimental.pallas import tpu as pltpu
from jax.experimental.pallas import tpu_sc as plsc

VOCAB = 1000000
EMBED_DIM = 64
BATCH = 16384

NC = 2
NS = 16
NW = NC * NS

L = 16
CHUNK = 128
IDX_PER_W = 2 * BATCH // NW
IDX_ROWS_PER_W = IDX_PER_W // CHUNK
N_PASS = 4
ROWS_PER_PASS = IDX_ROWS_PER_W // N_PASS
IDX_PER_PASS = ROWS_PER_PASS * CHUNK
OUT2_PER_PASS = IDX_PER_PASS // 2
OUT2_PER_W = IDX_PER_W // 2


def _do_work(pidx_hbm, mask_hbm, table2_hbm, out2_hbm, idx_v, pairs_v,
             mask_v, out_v, sem, k):
    pltpu.sync_copy(pidx_hbm.at[pl.ds(k * IDX_ROWS_PER_W, IDX_ROWS_PER_W)],
                    idx_v)
    for s in range(N_PASS):
        copies = [
            pltpu.async_copy(
                table2_hbm.at[idx_v.at[s * ROWS_PER_PASS + j]],
                pairs_v.at[pl.ds(j * CHUNK, CHUNK)],
                sem,
            )
            for j in range(ROWS_PER_PASS)
        ]
        out2_base = k * OUT2_PER_W + s * OUT2_PER_PASS
        pltpu.sync_copy(mask_hbm.at[pl.ds(out2_base, OUT2_PER_PASS)], mask_v)
        for c in copies:
            c.wait()

        @pl.loop(0, OUT2_PER_PASS // L)
        def _select(g):
            for rr in range(L):
                r = g * L + rr
                for c in range(8):
                    pr = 2 * r + (c // 4)
                    lo_off = (c % 4) * L
                    hi_off = 64 + (c % 4) * L
                    lo = pairs_v[pr, pl.ds(lo_off, L)]
                    hi = pairs_v[pr, pl.ds(hi_off, L)]
                    m = mask_v[r, pl.ds(c * L, L)]
                    out_v[r, pl.ds(c * L, L)] = lo + (hi - lo) * m

        pltpu.sync_copy(out_v, out2_hbm.at[pl.ds(out2_base, OUT2_PER_PASS)])


def _gather_body(pidx_t_hbm, pidx_o_hbm, mask_t_hbm, mask_o_hbm, table2_hbm,
                 out2_t_hbm, out2_o_hbm, idx_v, pairs_v, mask_v, out_v, sem):
    wid = lax.axis_index("s") * NC + lax.axis_index("c")

    @pl.when(wid < NW // 2)
    def _():
        _do_work(pidx_t_hbm, mask_t_hbm, table2_hbm, out2_t_hbm, idx_v,
                 pairs_v, mask_v, out_v, sem, wid)

    @pl.when(wid >= NW // 2)
    def _():
        _do_work(pidx_o_hbm, mask_o_hbm, table2_hbm, out2_o_hbm, idx_v,
                 pairs_v, mask_v, out_v, sem, wid - NW // 2)


@jax.jit
def _skipgram_gather(pidx_t, pidx_o, mask_t, mask_o, table2):
    mesh = plsc.VectorSubcoreMesh(core_axis_name="c", subcore_axis_name="s")
    out_sds = jax.ShapeDtypeStruct((BATCH // 2, 2 * EMBED_DIM), jnp.float32)
    run = pl.kernel(
        _gather_body,
        out_type=(out_sds, out_sds),
        mesh=mesh,
        compiler_params=pltpu.CompilerParams(use_tc_tiling_on_sc=True),
        scratch_types=[
            pltpu.VMEM((IDX_ROWS_PER_W, CHUNK), jnp.int32),
            pltpu.VMEM((IDX_PER_PASS, 2 * EMBED_DIM), jnp.float32),
            pltpu.VMEM((OUT2_PER_PASS, 2 * EMBED_DIM), jnp.float32),
            pltpu.VMEM((OUT2_PER_PASS, 2 * EMBED_DIM), jnp.float32),
            pltpu.SemaphoreType.DMA,
        ],
    )
    return run(pidx_t, pidx_o, mask_t, mask_o, table2)


def kernel(target, other, table):
    target = target.astype(jnp.int32)
    other = other.astype(jnp.int32)
    pidx_t = lax.shift_right_logical(target, 1).reshape(BATCH // CHUNK, CHUNK)
    pidx_o = lax.shift_right_logical(other, 1).reshape(BATCH // CHUNK, CHUNK)
    mask_t = jnp.broadcast_to(
        jnp.bitwise_and(target, 1).astype(jnp.float32)[:, None],
        (BATCH, EMBED_DIM)).reshape(BATCH // 2, 2 * EMBED_DIM)
    mask_o = jnp.broadcast_to(
        jnp.bitwise_and(other, 1).astype(jnp.float32)[:, None],
        (BATCH, EMBED_DIM)).reshape(BATCH // 2, 2 * EMBED_DIM)
    table2 = table.reshape(VOCAB // 2, 2 * EMBED_DIM)
    out2_t, out2_o = _skipgram_gather(pidx_t, pidx_o, mask_t, mask_o, table2)
    return (out2_t.reshape(BATCH, EMBED_DIM), out2_o.reshape(BATCH, EMBED_DIM))
